# Initial kernel scaffold; baseline (speedup 1.0000x reference)
import jax
import jax.numpy as jnp
from jax.experimental import pallas as pl


def kernel(x, W_enc, W_dec):
    raise NotImplementedError("write your pallas kernel here")


# speedup vs baseline: 17.8062x; 17.8062x over previous
import jax
import jax.numpy as jnp
from jax.experimental import pallas as pl
from jax.experimental.pallas import tpu as pltpu

_INPUT_DIM = 256
_BOTTLENECK = 1024
_K = 51
_ROWS = 256


def _fused_body(x_ref, we_ref, wd_ref, z2_ref, h1_ref):
    x = x_ref[...]
    z1 = jnp.dot(x, we_ref[...], preferred_element_type=jnp.float32)

    bits = jax.lax.bitcast_convert_type(z1, jnp.int32)
    ikey = bits ^ ((bits >> 31) & jnp.int32(0x7FFFFFFF))

    cnt = jnp.sum((ikey >= 0).astype(jnp.int32), axis=1, keepdims=True)
    t = jnp.where(cnt >= _K, jnp.int32(0), jnp.int32(-(2**31)))
    for b in range(30, -1, -1):
        cand = t + jnp.int32(1 << b)
        cnt = jnp.sum((ikey >= cand).astype(jnp.int32), axis=1, keepdims=True)
        t = jnp.where(cnt >= _K, cand, t)

    h1 = jnp.where(ikey >= t, z1, 0.0)
    h1_ref[...] = h1
    z2_ref[...] = jnp.dot(h1, wd_ref[...], preferred_element_type=jnp.float32)


def kernel(x, W_enc, W_dec):
    if x.ndim == 1:
        x = x[None, :]
    batch = x.shape[0]
    rows = _ROWS
    pad = (-batch) % rows
    xp = jnp.pad(x, ((0, pad), (0, 0))) if pad else x
    nblocks = xp.shape[0] // rows

    we_t = W_enc.T
    wd_t = W_dec.T

    z2, h1 = pl.pallas_call(
        _fused_body,
        grid=(nblocks,),
        in_specs=[
            pl.BlockSpec((rows, _INPUT_DIM), lambda i: (i, 0)),
            pl.BlockSpec((_INPUT_DIM, _BOTTLENECK), lambda i: (0, 0)),
            pl.BlockSpec((_BOTTLENECK, _INPUT_DIM), lambda i: (0, 0)),
        ],
        out_specs=[
            pl.BlockSpec((rows, _INPUT_DIM), lambda i: (i, 0)),
            pl.BlockSpec((rows, _BOTTLENECK), lambda i: (i, 0)),
        ],
        out_shape=[
            jax.ShapeDtypeStruct((xp.shape[0], _INPUT_DIM), jnp.float32),
            jax.ShapeDtypeStruct((xp.shape[0], _BOTTLENECK), jnp.float32),
        ],
        compiler_params=pltpu.CompilerParams(
            dimension_semantics=("arbitrary",),
        ),
    )(xp, we_t, wd_t)

    if pad:
        z2 = z2[:batch]
        h1 = h1[:batch]
    return (z2, h1)
